# baseline (speedup 1.0000x reference)
import functools

import jax
import jax.numpy as jnp
from jax import lax
from jax.experimental import pallas as pl
from jax.experimental.pallas import tpu as pltpu
from jax.experimental.pallas import tpu_sc as plsc

VOCAB = 30522
DIM = 768
SEQ = 512
BATCH = 64
EPS = 1e-12

ROWS = BATCH * SEQ
NLANE = 16
NVREG = DIM // NLANE
CHUNK = 64

_info = plsc.get_sparse_core_info()
NC = _info.num_cores
NS = _info.num_subcores
NW = NC * NS
ROWS_W = ROWS // NW
NCHUNK = ROWS_W // CHUNK

_mesh = plsc.VectorSubcoreMesh(core_axis_name="c", subcore_axis_name="s")

_GDN = lax.GatherDimensionNumbers(
    offset_dims=(), collapsed_slice_dims=(0,), start_index_map=(0,))


def _lane_allreduce(x):
    lanes = jnp.arange(NLANE, dtype=jnp.int32)
    for k in (1, 2, 4, 8):
        perm = (lanes ^ k).reshape(NLANE, 1)
        x = x + lax.gather(x, perm, _GDN, (1,),
                           mode=lax.GatherScatterMode.PROMISE_IN_BOUNDS)
    return x


@functools.partial(
    pl.kernel,
    mesh=_mesh,
    out_type=jax.ShapeDtypeStruct((ROWS, DIM), jnp.float32),
    scratch_types=[
        pltpu.VMEM((CHUNK,), jnp.int32),
        pltpu.VMEM((CHUNK, DIM), jnp.float32),
        pltpu.VMEM((CHUNK, DIM), jnp.float32),
        pltpu.VMEM((DIM,), jnp.float32),
        pltpu.VMEM((DIM,), jnp.float32),
        pltpu.SemaphoreType.DMA,
    ],
)
def _ln_embed(word_hbm, idx_hbm, pos_hbm, gamma_hbm, beta_hbm, out_hbm,
              idx_v, rows_v, pos_v, gamma_v, beta_v, sem):
    wid = lax.axis_index("s") * NC + lax.axis_index("c")
    slab = wid * ROWS_W

    pltpu.sync_copy(gamma_hbm, gamma_v)
    pltpu.sync_copy(beta_hbm, beta_v)

    def chunk_body(c, carry):
        base = slab + c * CHUNK
        s0 = (c % (SEQ // CHUNK)) * CHUNK
        pltpu.sync_copy(idx_hbm.at[pl.ds(base, CHUNK)], idx_v)
        pltpu.async_copy(word_hbm.at[idx_v], rows_v, sem).wait()
        pltpu.sync_copy(pos_hbm.at[pl.ds(s0, CHUNK)], pos_v)

        def row_body(i, rcarry):
            ssum = jnp.zeros((NLANE,), jnp.float32)
            ssq = jnp.zeros((NLANE,), jnp.float32)
            for j in range(NVREG):
                sl = pl.ds(j * NLANE, NLANE)
                x = rows_v[i, sl] + pos_v[i, sl]
                rows_v[i, sl] = x
                ssum = ssum + x
                ssq = ssq + x * x
            mean_v = _lane_allreduce(ssum) * (1.0 / DIM)
            var_v = _lane_allreduce(ssq) * (1.0 / DIM) - mean_v * mean_v
            v = var_v + EPS
            bits = lax.bitcast_convert_type(v, jnp.int32)
            y = lax.bitcast_convert_type(
                jnp.int32(0x5F3759DF) - (bits >> 1), jnp.float32)
            for _ in range(3):
                y = y * (1.5 - 0.5 * v * y * y)
            rstd = y
            for j in range(NVREG):
                sl = pl.ds(j * NLANE, NLANE)
                a = gamma_v[sl] * rstd
                rows_v[i, sl] = (rows_v[i, sl] - mean_v) * a + beta_v[sl]
            return rcarry

        lax.fori_loop(0, CHUNK, row_body, 0)
        pltpu.sync_copy(rows_v, out_hbm.at[pl.ds(base, CHUNK)])
        return carry

    lax.fori_loop(0, NCHUNK, chunk_body, 0)


def kernel(news_batch, word_table, pos_table, gamma, beta):
    idx = news_batch.reshape(ROWS).astype(jnp.int32)
    out = _ln_embed(word_table, idx, pos_table, gamma, beta)
    return out.reshape(BATCH, SEQ, DIM)
